# baseline (speedup 1.0000x reference)
import functools

import jax
import jax.numpy as jnp
from jax import lax
from jax.experimental import pallas as pl
from jax.experimental.pallas import tpu as pltpu
from jax.experimental.pallas import tpu_sc as plsc

_N = 50000
_D = 64
_DH = 32
_E = 800000
_NS = 16
_EPT = _E // _NS
_C = 80
_CH = 25
_NSUB = _EPT // (_CH * _C)
_NP = 50048
_RPT = _NP // _NS
_ZC = 136
_NZ = _RPT // _ZC

_f32 = jnp.float32
_bf16 = jnp.bfloat16
_i32 = jnp.int32

_INTER = plsc.PackFormat.INTERLEAVED


def _body(row_hbm, col_hbm, val_hbm, u0, i0, out_u, out_i,
          t1, t2, s1, s2,
          acc, col2d, dst2d, val2d, rows0, rows1, rows2, rows3,
          sb0, sb1, zwf, bw, b1, b2,
          sem_g0, sem_g1, sem_g2, sem_g3, sem_s0, sem_s1):
    h = lax.axis_index("c")
    tid = lax.axis_index("s")
    hoff = jnp.full((16,), h * _NP, _i32)

    def spmm(src, dst, fold=None):
        def zinit(r, c):
            z16 = jnp.zeros((16,), _f32)
            zwf[r, pl.ds(0, 16)] = z16
            zwf[r, pl.ds(16, 16)] = z16
            return c
        lax.fori_loop(0, _ZC, zinit, 0)

        def zero_chunk(z, c):
            pltpu.sync_copy(zwf, acc.at[pl.ds(tid * _RPT + z * _ZC, _ZC)])
            return c
        lax.fori_loop(0, _NZ, zero_chunk, 0)
        plsc.subcore_barrier()

        rbufs = ((rows0, sem_g0), (rows1, sem_g1),
                 (rows2, sem_g2), (rows3, sem_g3))
        sbufs = ((sb0, sem_s0), (sb1, sem_s1))

        def scale(b, sb, j):
            rows = rbufs[b][0]
            sbuf = sbufs[sb][0]

            def scale_group(g, c):
                vv = val2d[j, pl.ds(g * 16, 16)]
                lane = jnp.zeros((16,), _i32)
                one = jnp.full((16,), 1, _i32)
                for l in range(16):
                    e = g * 16 + l
                    sp = lax.gather(
                        vv, lane.reshape(16, 1),
                        lax.GatherDimensionNumbers(
                            offset_dims=(), collapsed_slice_dims=(0,),
                            start_index_map=(0,)),
                        (1,), mode=lax.GatherScatterMode.PROMISE_IN_BOUNDS)
                    xa, xb = plsc.unpack(rows[e, :], format=_INTER)
                    sbuf[e, pl.ds(0, 16)] = xa * sp
                    sbuf[e, pl.ds(16, 16)] = xb * sp
                    lane = lane + one
                return c
            lax.fori_loop(0, _C // 16, scale_group, 0)

        def wait_scat(sb, j):
            sbuf, sem = sbufs[sb]
            pltpu.make_async_copy(sbuf, acc.at[dst2d.at[j]], sem).wait()

        def comp(b, sb, j):
            rows, sem_g = rbufs[b]
            sbuf, sem_s = sbufs[sb]
            pltpu.make_async_copy(src.at[col2d.at[j]], rows, sem_g).wait()
            scale(b, sb, j)
            pltpu.async_copy(sbuf, acc.at[dst2d.at[j]], sem_s, add=True)

        def gath(b, j):
            rows, sem_g = rbufs[b]
            pltpu.async_copy(src.at[col2d.at[j]], rows, sem_g)

        def subpass(s, c):
            r0 = tid * (_CH * _NSUB) + s * _CH
            pltpu.sync_copy(col_hbm.at[pl.ds(r0, _CH)], col2d)
            pltpu.sync_copy(row_hbm.at[pl.ds(r0, _CH)], dst2d)
            pltpu.sync_copy(val_hbm.at[pl.ds(r0, _CH)], val2d)

            def adjust(r, cc):
                for g in range(_C // 16):
                    col2d[r, pl.ds(g * 16, 16)] = (
                        col2d[r, pl.ds(g * 16, 16)] + hoff)
                return cc
            lax.fori_loop(0, _CH, adjust, 0)

            gath(0, 0)
            gath(1, 1)

            def quad(q, cc):
                for b in range(4):
                    j = 4 * q + b
                    @pl.when(j + 2 < _CH)
                    def _():
                        gath((b + 2) % 4, j + 2)
                    if b >= 2:
                        wait_scat(b % 2, j - 2)
                    else:
                        @pl.when(q > 0)
                        def _():
                            wait_scat(b % 2, j - 2)
                    comp(b, b % 2, j)
                return cc
            lax.fori_loop(0, _CH // 4, quad, 0)

            wait_scat((_CH - 3) % 2, _CH - 3)
            comp(0, (_CH - 1) % 2, _CH - 1)
            wait_scat((_CH - 2) % 2, _CH - 2)
            wait_scat((_CH - 1) % 2, _CH - 1)
            return c
        lax.fori_loop(0, _NSUB, subpass, 0)
        plsc.subcore_barrier()

        def wb_chunk(z, c):
            r0 = tid * _RPT + z * _ZC
            pltpu.sync_copy(acc.at[pl.ds(r0, _ZC)], zwf)
            if fold is not None:
                fa, fb = fold
                pltpu.sync_copy(fa.at[pl.ds(h * _NP + r0, _ZC)], b1)
                pltpu.sync_copy(fb.at[pl.ds(h * _NP + r0, _ZC)], b2)

                def fold_row(r, cc):
                    third = jnp.full((16,), 1.0 / 3.0, _f32)
                    pa, pb = plsc.unpack(b1[r, :], format=_INTER)
                    qa, qb = plsc.unpack(b2[r, :], format=_INTER)
                    zwf[r, pl.ds(0, 16)] = (
                        zwf[r, pl.ds(0, 16)] + pa + qa) * third
                    zwf[r, pl.ds(16, 16)] = (
                        zwf[r, pl.ds(16, 16)] + pb + qb) * third
                    return cc
                lax.fori_loop(0, _ZC, fold_row, 0)
                pltpu.sync_copy(zwf, dst.at[pl.ds(h * _NP + r0, _ZC)])
            else:
                def pack_row(r, cc):
                    bw[r, :] = plsc.pack(zwf[r, pl.ds(0, 16)],
                                         zwf[r, pl.ds(16, 16)],
                                         format=_INTER)
                    return cc
                lax.fori_loop(0, _ZC, pack_row, 0)
                pltpu.sync_copy(bw, dst.at[pl.ds(h * _NP + r0, _ZC)])
            return c
        lax.fori_loop(0, _NZ, wb_chunk, 0)
        plsc.subcore_barrier()

    spmm(i0, t1)
    spmm(u0, s1)
    spmm(s1, t2)
    spmm(t1, s2)
    spmm(s2, out_u, fold=(t1, t2))
    spmm(t2, out_i, fold=(s1, s2))


_sds = jax.ShapeDtypeStruct

_gcn = functools.partial(
    pl.kernel,
    out_type=(_sds((2 * _NP, _DH), _f32), _sds((2 * _NP, _DH), _f32)),
    mesh=plsc.VectorSubcoreMesh(core_axis_name="c", subcore_axis_name="s"),
    compiler_params=pltpu.CompilerParams(use_tc_tiling_on_sc=False,
                                         needs_layout_passes=False),
    scratch_types=[
        pltpu.HBM((2 * _NP, _DH), _bf16),
        pltpu.HBM((2 * _NP, _DH), _bf16),
        pltpu.HBM((2 * _NP, _DH), _bf16),
        pltpu.HBM((2 * _NP, _DH), _bf16),
        pltpu.VMEM_SHARED((_NP, _DH), _f32),
        pltpu.VMEM((_CH, _C), _i32),
        pltpu.VMEM((_CH, _C), _i32),
        pltpu.VMEM((_CH, _C), _f32),
        pltpu.VMEM((_C, _DH), _bf16),
        pltpu.VMEM((_C, _DH), _bf16),
        pltpu.VMEM((_C, _DH), _bf16),
        pltpu.VMEM((_C, _DH), _bf16),
        pltpu.VMEM((_C, _DH), _f32),
        pltpu.VMEM((_C, _DH), _f32),
        pltpu.VMEM((_ZC, _DH), _f32),
        pltpu.VMEM((_ZC, _DH), _bf16),
        pltpu.VMEM((_ZC, _DH), _bf16),
        pltpu.VMEM((_ZC, _DH), _bf16),
        pltpu.SemaphoreType.DMA,
        pltpu.SemaphoreType.DMA,
        pltpu.SemaphoreType.DMA,
        pltpu.SemaphoreType.DMA,
        pltpu.SemaphoreType.DMA,
        pltpu.SemaphoreType.DMA,
    ],
)(_body)


def kernel(user_embeddings, item_embeddings, edge_index, edge_values):
    row = edge_index[0].astype(_i32).reshape(_E // _C, _C)
    col = edge_index[1].astype(_i32).reshape(_E // _C, _C)
    val2 = edge_values.reshape(_E // _C, _C)

    def stack(t):
        s = jnp.zeros((2 * _NP, _DH), _bf16)
        t = t.astype(_bf16)
        return s.at[:_N].set(t[:, :_DH]).at[_NP:_NP + _N].set(t[:, _DH:])
    out_u, out_i = _gcn(row, col, val2,
                        stack(user_embeddings), stack(item_embeddings))
    pos = jnp.array([c // 2 if c % 2 == 0 else 16 + c // 2
                     for c in range(_DH)], _i32)

    def unstack(o):
        return jnp.concatenate([o[:_N][:, pos], o[_NP:_NP + _N][:, pos]],
                               axis=1)
    return (unstack(out_u), unstack(out_i))
